# Initial kernel scaffold; baseline (speedup 1.0000x reference)
import jax
import jax.numpy as jnp
from jax.experimental import pallas as pl


def kernel(input_x, G_rows, G_cols, G_vals, E_rows, E_cols, E_vals, W0, b0, W1, b1):
    raise NotImplementedError("write your pallas kernel here")


# speedup vs baseline: 4.0564x; 4.0564x over previous
import functools

import jax
import jax.numpy as jnp
from jax import lax
from jax.experimental import pallas as pl
from jax.experimental.pallas import tpu as pltpu
from jax.experimental.pallas import tpu_sc as plsc

NC = 2
NS = 16
NW = NC * NS
LANES = 16
P_DROP = 0.1
M_EDGES = 5000


def _sc_spmm(rows, cols, vals, x, num_rows, k):
    nnz = rows.shape[0]
    n, d = x.shape
    chunk = NW * k
    if nnz % chunk:
        pad = chunk - nnz % chunk
        rows = jnp.concatenate([rows, jnp.zeros((pad,), jnp.int32)])
        cols = jnp.concatenate([cols, jnp.zeros((pad,), jnp.int32)])
        vals = jnp.concatenate([vals, jnp.zeros((pad,), jnp.float32)])
        nnz += pad
    nnz_w = nnz // NW
    nb = nnz_w // k
    assert nb * k == nnz_w and k % 16 == 0 and k <= 128
    assert num_rows % 8 == 0
    rpt = (num_rows // (NS * 8)) * 8
    rem = num_rows - rpt * NS
    mesh = plsc.VectorSubcoreMesh(core_axis_name="c", subcore_axis_name="s")

    @functools.partial(
        pl.kernel,
        out_type=jax.ShapeDtypeStruct((NC, num_rows, d), jnp.float32),
        mesh=mesh,
        scratch_types=[
            pltpu.VMEM((k,), jnp.int32),
            pltpu.VMEM((k,), jnp.int32),
            pltpu.VMEM((k,), jnp.float32),
            pltpu.VMEM((k, d), jnp.float32),
            pltpu.VMEM_SHARED((num_rows, d), jnp.float32),
            pltpu.SemaphoreType.DMA,
        ],
    )
    def spmm(rows_hbm, cols_hbm, vals_hbm, x_hbm, zeros_hbm, out_hbm,
             rows_v, cols_v, vals_v, xbuf, acc, sem):
        c = lax.axis_index("c")
        s = lax.axis_index("s")

        pltpu.sync_copy(zeros_hbm.at[pl.ds(s * rpt, rpt)],
                        acc.at[pl.ds(s * rpt, rpt)])
        if rem:
            @pl.when(s == NS - 1)
            def _():
                pltpu.sync_copy(zeros_hbm.at[pl.ds(NS * rpt, rem)],
                                acc.at[pl.ds(NS * rpt, rem)])
        plsc.subcore_barrier()

        w = s * NC + c
        base0 = w * nnz_w

        def body(b, carry):
            base = base0 + b * k
            pltpu.sync_copy(rows_hbm.at[pl.ds(base, k)], rows_v)
            pltpu.sync_copy(cols_hbm.at[pl.ds(base, k)], cols_v)
            pltpu.sync_copy(vals_hbm.at[pl.ds(base, k)], vals_v)
            pltpu.async_copy(x_hbm.at[cols_v], xbuf, sem).wait()

            def scale(g, carry2):
                vv = vals_v[pl.ds(g * LANES, LANES)]
                for i2 in range(LANES):
                    v = vv[i2]
                    row = g * LANES + i2
                    for j in range(d // LANES):
                        sl = pl.ds(j * LANES, LANES)
                        xbuf[row, sl] = xbuf[row, sl] * v
                return carry2

            lax.fori_loop(0, k // LANES, scale, 0)
            pltpu.sync_copy(xbuf, acc.at[rows_v], add=True)
            return carry

        lax.fori_loop(0, nb, body, 0)
        plsc.subcore_barrier()

        pltpu.sync_copy(acc.at[pl.ds(s * rpt, rpt)],
                        out_hbm.at[c, pl.ds(s * rpt, rpt)])
        if rem:
            @pl.when(s == NS - 1)
            def _():
                pltpu.sync_copy(acc.at[pl.ds(NS * rpt, rem)],
                                out_hbm.at[c, pl.ds(NS * rpt, rem)])

    zeros = jnp.zeros((num_rows, d), jnp.float32)
    return spmm(rows, cols, vals, x, zeros)


def _tc_matmul(x, w, b):
    n, d = x.shape
    bs = 2000 if n % 2000 == 0 else 1000

    def body(x_ref, w_ref, b_ref, o_ref):
        o_ref[...] = (jnp.dot(x_ref[...], w_ref[...],
                              preferred_element_type=jnp.float32)
                      + b_ref[...])

    return pl.pallas_call(
        body,
        grid=(n // bs,),
        in_specs=[pl.BlockSpec((bs, d), lambda i: (i, 0)),
                  pl.BlockSpec((d, d), lambda i: (0, 0)),
                  pl.BlockSpec((1, d), lambda i: (0, 0))],
        out_specs=pl.BlockSpec((bs, d), lambda i: (i, 0)),
        out_shape=jax.ShapeDtypeStruct((n, d), jnp.float32),
    )(x, w, b.reshape(1, d))


def _tc_combine_drop_matmul(parts, scale, w, b):
    _, n, d = parts.shape
    bs = 2000 if n % 2000 == 0 else 1000

    def body(p_ref, s_ref, w_ref, b_ref, o_ref):
        h = jnp.maximum(p_ref[0] + p_ref[1], 0.0) * s_ref[...]
        o_ref[...] = (jnp.dot(h, w_ref[...],
                              preferred_element_type=jnp.float32)
                      + b_ref[...])

    return pl.pallas_call(
        body,
        grid=(n // bs,),
        in_specs=[pl.BlockSpec((2, bs, d), lambda i: (0, i, 0)),
                  pl.BlockSpec((bs, d), lambda i: (i, 0)),
                  pl.BlockSpec((d, d), lambda i: (0, 0)),
                  pl.BlockSpec((1, d), lambda i: (0, 0))],
        out_specs=pl.BlockSpec((bs, d), lambda i: (i, 0)),
        out_shape=jax.ShapeDtypeStruct((n, d), jnp.float32),
    )(parts, scale, w, b.reshape(1, d))


def _tc_combine_and_drop(parts, scale):
    _, n, d = parts.shape
    bs = 2000 if n % 2000 == 0 else 1000

    def body(p_ref, s_ref, o_ref, o2_ref):
        h = jnp.maximum(p_ref[0] + p_ref[1], 0.0)
        o_ref[...] = h
        o2_ref[...] = h * s_ref[...]

    return pl.pallas_call(
        body,
        grid=(n // bs,),
        in_specs=[pl.BlockSpec((2, bs, d), lambda i: (0, i, 0)),
                  pl.BlockSpec((bs, d), lambda i: (i, 0))],
        out_specs=[pl.BlockSpec((bs, d), lambda i: (i, 0)),
                   pl.BlockSpec((bs, d), lambda i: (i, 0))],
        out_shape=[jax.ShapeDtypeStruct((n, d), jnp.float32),
                   jax.ShapeDtypeStruct((n, d), jnp.float32)],
    )(parts, scale)


def _tc_combine_relu(parts):
    _, n, d = parts.shape
    bs = 2000 if n % 2000 == 0 else 1000

    def body(p_ref, o_ref):
        o_ref[...] = jnp.maximum(p_ref[0] + p_ref[1], 0.0)

    return pl.pallas_call(
        body,
        grid=(n // bs,),
        in_specs=[pl.BlockSpec((2, bs, d), lambda i: (0, i, 0))],
        out_specs=pl.BlockSpec((bs, d), lambda i: (i, 0)),
        out_shape=jax.ShapeDtypeStruct((n, d), jnp.float32),
    )(parts)


def kernel(input_x, G_rows, G_cols, G_vals, E_rows, E_cols, E_vals,
           W0, b0, W1, b1):
    n, d = input_x.shape

    dk = jax.random.key(42)
    keep1 = jax.random.bernoulli(jax.random.fold_in(dk, 1), 1.0 - P_DROP,
                                 (n, d))
    keep2 = jax.random.bernoulli(jax.random.fold_in(dk, 2), 1.0 - P_DROP,
                                 (n, d))
    s1 = keep1.astype(jnp.float32) / (1.0 - P_DROP)
    s2 = keep2.astype(jnp.float32) / (1.0 - P_DROP)

    x0 = _tc_matmul(input_x, W0, b0)
    parts1 = _sc_spmm(G_rows, G_cols, G_vals, x0, n, 80)
    x1 = _tc_combine_drop_matmul(parts1, s1, W1, b1)
    parts2 = _sc_spmm(G_rows, G_cols, G_vals, x1, n, 80)
    nodes, dropped = _tc_combine_and_drop(parts2, s2)
    parts3 = _sc_spmm(E_rows, E_cols, E_vals, dropped, M_EDGES, 80)
    edges = _tc_combine_relu(parts3)
    return (nodes, edges)


# speedup vs baseline: 6.9581x; 1.7154x over previous
import functools

import jax
import jax.numpy as jnp
from jax import lax
from jax.experimental import pallas as pl
from jax.experimental.pallas import tpu as pltpu
from jax.experimental.pallas import tpu_sc as plsc

NC = 2
NS = 16
NW = NC * NS
LANES = 16
P_DROP = 0.1
M_EDGES = 5000


def _sc_spmm(rows, cols, vals, x, num_rows, k):
    nnz = rows.shape[0]
    n, d = x.shape
    chunk = NW * k * 3
    if nnz % chunk:
        pad = chunk - nnz % chunk
        rows = jnp.concatenate([rows, jnp.zeros((pad,), jnp.int32)])
        cols = jnp.concatenate([cols, jnp.zeros((pad,), jnp.int32)])
        vals = jnp.concatenate([vals, jnp.zeros((pad,), jnp.float32)])
        nnz += pad
    nnz_w = nnz // NW
    nb = nnz_w // k
    assert nb * k == nnz_w and nb % 3 == 0 and k % 16 == 0 and k <= 128
    rows3 = rows.reshape(NW, nb, k)
    cols3 = cols.reshape(NW, nb, k)
    vals3 = vals.reshape(NW, nb, k)
    assert num_rows % 8 == 0
    rpt = (num_rows // (NS * 8)) * 8
    rem = num_rows - rpt * NS
    mesh = plsc.VectorSubcoreMesh(core_axis_name="c", subcore_axis_name="s")

    @functools.partial(
        pl.kernel,
        out_type=jax.ShapeDtypeStruct((NC, num_rows, d), jnp.float32),
        mesh=mesh,
        scratch_types=[
            pltpu.VMEM((nb, k), jnp.int32),
            pltpu.VMEM((3, k), jnp.int32),
            pltpu.VMEM((3, k), jnp.float32),
            pltpu.VMEM((k, d), jnp.float32),
            pltpu.VMEM((k, d), jnp.float32),
            pltpu.VMEM((k, d), jnp.float32),
            pltpu.VMEM_SHARED((num_rows, d), jnp.float32),
            [pltpu.SemaphoreType.DMA] * 3,
            [pltpu.SemaphoreType.DMA] * 3,
            [pltpu.SemaphoreType.DMA] * 3,
        ],
    )
    def spmm(rows_hbm, cols_hbm, vals_hbm, x_hbm, zeros_hbm, out_hbm,
             colp, rring, vring, gb0, gb1, gb2, acc,
             rvsem, gsem, ssem):
        c = lax.axis_index("c")
        s = lax.axis_index("s")
        w = s * NC + c
        gbufs = (gb0, gb1, gb2)

        pltpu.sync_copy(cols_hbm.at[w], colp)

        pltpu.sync_copy(zeros_hbm.at[pl.ds(s * rpt, rpt)],
                        acc.at[pl.ds(s * rpt, rpt)])
        if rem:
            @pl.when(s == NS - 1)
            def _():
                pltpu.sync_copy(zeros_hbm.at[pl.ds(NS * rpt, rem)],
                                acc.at[pl.ds(NS * rpt, rem)])
        plsc.subcore_barrier()

        def rv_start(b, r):
            pltpu.async_copy(rows_hbm.at[w, b], rring.at[r], rvsem[r])
            pltpu.async_copy(vals_hbm.at[w, b], vring.at[r], rvsem[r])

        def rv_wait(r):
            pltpu.make_async_copy(rows_hbm.at[0, 0], rring.at[r],
                                  rvsem[r]).wait()
            pltpu.make_async_copy(vals_hbm.at[0, 0], vring.at[r],
                                  rvsem[r]).wait()

        def gather_start(b, r):
            pltpu.async_copy(x_hbm.at[colp.at[b]], gbufs[r], gsem[r])

        def gather_wait(r):
            pltpu.make_async_copy(x_hbm.at[pl.ds(0, k)], gbufs[r],
                                  gsem[r]).wait()

        def scatter_start(r):
            pltpu.async_copy(gbufs[r], acc.at[rring.at[r]], ssem[r],
                             add=True)

        def scatter_wait(r):
            pltpu.make_async_copy(gbufs[r], acc.at[pl.ds(0, k)],
                                  ssem[r]).wait()

        def scale(r):
            xbuf = gbufs[r]

            def group(g, carry2):
                vv = vring[r, pl.ds(g * LANES, LANES)]
                for i2 in range(LANES):
                    v = vv[i2]
                    row = g * LANES + i2
                    for jj in range(d // LANES):
                        sl = pl.ds(jj * LANES, LANES)
                        xbuf[row, sl] = xbuf[row, sl] * v
                return carry2

            lax.fori_loop(0, k // LANES, group, 0)

        for j in range(2):
            rv_start(j, j)
        gather_start(0, 0)
        gather_start(1, 1)

        def body(u, carry):
            for i in range(3):
                b = 3 * u + i
                r = i
                r2 = (i + 2) % 3
                gather_wait(r)
                rv_wait(r)
                scale(r)
                scatter_start(r)

                @pl.when(b + 2 < nb)
                def _():
                    @pl.when(b >= 1)
                    def _():
                        scatter_wait(r2)

                    rv_start(b + 2, r2)
                    gather_start(b + 2, r2)

            return carry

        lax.fori_loop(0, nb // 3, body, 0)
        scatter_wait(0)
        scatter_wait(1)
        scatter_wait(2)
        plsc.subcore_barrier()

        pltpu.sync_copy(acc.at[pl.ds(s * rpt, rpt)],
                        out_hbm.at[c, pl.ds(s * rpt, rpt)])
        if rem:
            @pl.when(s == NS - 1)
            def _():
                pltpu.sync_copy(acc.at[pl.ds(NS * rpt, rem)],
                                out_hbm.at[c, pl.ds(NS * rpt, rem)])

    zeros = jnp.zeros((num_rows, d), jnp.float32)
    return spmm(rows3, cols3, vals3, x, zeros)


def _tc_matmul(x, w, b):
    n, d = x.shape
    bs = 2000 if n % 2000 == 0 else 1000

    def body(x_ref, w_ref, b_ref, o_ref):
        o_ref[...] = (jnp.dot(x_ref[...], w_ref[...],
                              preferred_element_type=jnp.float32)
                      + b_ref[...])

    return pl.pallas_call(
        body,
        grid=(n // bs,),
        in_specs=[pl.BlockSpec((bs, d), lambda i: (i, 0)),
                  pl.BlockSpec((d, d), lambda i: (0, 0)),
                  pl.BlockSpec((1, d), lambda i: (0, 0))],
        out_specs=pl.BlockSpec((bs, d), lambda i: (i, 0)),
        out_shape=jax.ShapeDtypeStruct((n, d), jnp.float32),
    )(x, w, b.reshape(1, d))


def _tc_combine_drop_matmul(parts, scale, w, b):
    _, n, d = parts.shape
    bs = 2000 if n % 2000 == 0 else 1000

    def body(p_ref, s_ref, w_ref, b_ref, o_ref):
        h = jnp.maximum(p_ref[0] + p_ref[1], 0.0) * s_ref[...]
        o_ref[...] = (jnp.dot(h, w_ref[...],
                              preferred_element_type=jnp.float32)
                      + b_ref[...])

    return pl.pallas_call(
        body,
        grid=(n // bs,),
        in_specs=[pl.BlockSpec((2, bs, d), lambda i: (0, i, 0)),
                  pl.BlockSpec((bs, d), lambda i: (i, 0)),
                  pl.BlockSpec((d, d), lambda i: (0, 0)),
                  pl.BlockSpec((1, d), lambda i: (0, 0))],
        out_specs=pl.BlockSpec((bs, d), lambda i: (i, 0)),
        out_shape=jax.ShapeDtypeStruct((n, d), jnp.float32),
    )(parts, scale, w, b.reshape(1, d))


def _tc_combine_and_drop(parts, scale):
    _, n, d = parts.shape
    bs = 2000 if n % 2000 == 0 else 1000

    def body(p_ref, s_ref, o_ref, o2_ref):
        h = jnp.maximum(p_ref[0] + p_ref[1], 0.0)
        o_ref[...] = h
        o2_ref[...] = h * s_ref[...]

    return pl.pallas_call(
        body,
        grid=(n // bs,),
        in_specs=[pl.BlockSpec((2, bs, d), lambda i: (0, i, 0)),
                  pl.BlockSpec((bs, d), lambda i: (i, 0))],
        out_specs=[pl.BlockSpec((bs, d), lambda i: (i, 0)),
                   pl.BlockSpec((bs, d), lambda i: (i, 0))],
        out_shape=[jax.ShapeDtypeStruct((n, d), jnp.float32),
                   jax.ShapeDtypeStruct((n, d), jnp.float32)],
    )(parts, scale)


def _tc_combine_relu(parts):
    _, n, d = parts.shape
    bs = 2000 if n % 2000 == 0 else 1000

    def body(p_ref, o_ref):
        o_ref[...] = jnp.maximum(p_ref[0] + p_ref[1], 0.0)

    return pl.pallas_call(
        body,
        grid=(n // bs,),
        in_specs=[pl.BlockSpec((2, bs, d), lambda i: (0, i, 0))],
        out_specs=pl.BlockSpec((bs, d), lambda i: (i, 0)),
        out_shape=jax.ShapeDtypeStruct((n, d), jnp.float32),
    )(parts)


def kernel(input_x, G_rows, G_cols, G_vals, E_rows, E_cols, E_vals,
           W0, b0, W1, b1):
    n, d = input_x.shape

    dk = jax.random.key(42)
    keep1 = jax.random.bernoulli(jax.random.fold_in(dk, 1), 1.0 - P_DROP,
                                 (n, d))
    keep2 = jax.random.bernoulli(jax.random.fold_in(dk, 2), 1.0 - P_DROP,
                                 (n, d))
    s1 = keep1.astype(jnp.float32) / (1.0 - P_DROP)
    s2 = keep2.astype(jnp.float32) / (1.0 - P_DROP)

    x0 = _tc_matmul(input_x, W0, b0)
    parts1 = _sc_spmm(G_rows, G_cols, G_vals, x0, n, 80)
    x1 = _tc_combine_drop_matmul(parts1, s1, W1, b1)
    parts2 = _sc_spmm(G_rows, G_cols, G_vals, x1, n, 80)
    nodes, dropped = _tc_combine_and_drop(parts2, s2)
    parts3 = _sc_spmm(E_rows, E_cols, E_vals, dropped, M_EDGES, 80)
    edges = _tc_combine_relu(parts3)
    return (nodes, edges)
